# baseline (speedup 1.0000x reference)
import functools
import math

import jax
import jax.numpy as jnp
from jax import lax
from jax.experimental import pallas as pl
from jax.experimental.pallas import tpu as pltpu
from jax.experimental.pallas import tpu_sc as plsc

D = 768
H = 12
HD = 64
E = 8
HID = 3072
B = 8
NV = 197
NP = 200
T = B * NP
TV = B * NV
C = int(math.ceil(TV * 1.0 / E * 1.25))
CP = 256
EC = E * CP
TRASH = EC
HC = 768
F32 = jnp.float32
BF16 = jnp.bfloat16


def _ln(x, g, b):
    m = jnp.mean(x, axis=-1, keepdims=True)
    v = jnp.mean((x - m) ** 2, axis=-1, keepdims=True)
    return (x - m) / jnp.sqrt(v + 1e-6) * g + b


def _bdot(a, b):
    return lax.dot_general(a.astype(BF16), b.astype(BF16),
                           (((1,), (0,)), ((), ())),
                           preferred_element_type=F32)


def _embed_krn(p_ref, pw_ref, pb_ref, cls_ref, pos_ref, out_ref):
    body = _bdot(p_ref[0], pw_ref[...]) + pb_ref[...]
    full = jnp.concatenate(
        [cls_ref[...], body, jnp.zeros((NP - NV, D), F32)], axis=0)
    out_ref[0] = full + pos_ref[...]


def _embed(patches, pw, pb, cls_vec, pos_pad):
    return pl.pallas_call(
        _embed_krn,
        grid=(B,),
        in_specs=[
            pl.BlockSpec((1, NV - 1, D), lambda b: (b, 0, 0)),
            pl.BlockSpec((D, D), lambda b: (0, 0)),
            pl.BlockSpec((D,), lambda b: (0,)),
            pl.BlockSpec((1, D), lambda b: (0, 0)),
            pl.BlockSpec((NP, D), lambda b: (0, 0)),
        ],
        out_specs=pl.BlockSpec((1, NP, D), lambda b: (b, 0, 0)),
        out_shape=jax.ShapeDtypeStruct((B, NP, D), F32),
    )(patches, pw, pb, cls_vec, pos_pad)


def _attn_krn(fuse_y, *refs):
    if fuse_y:
        (t_ref, y_ref, val_ref, g_ref, b_ref, qw_ref, qb_ref,
         pw_ref, pb_ref, out_ref) = refs
        x = t_ref[0] + y_ref[0] * val_ref[0, 0][:, None]
    else:
        (t_ref, g_ref, b_ref, qw_ref, qb_ref, pw_ref, pb_ref,
         out_ref) = refs
        x = t_ref[0]
    xln = _ln(x, g_ref[...], b_ref[...])
    qkv = _bdot(xln, qw_ref[...]) + qb_ref[...]
    col = lax.broadcasted_iota(jnp.int32, (NP, NP), 1)
    outs = []
    for h in range(H):
        q = qkv[:, h * HD:(h + 1) * HD]
        k = qkv[:, D + h * HD:D + (h + 1) * HD]
        v = qkv[:, 2 * D + h * HD:2 * D + (h + 1) * HD]
        s = lax.dot_general(q.astype(BF16), k.astype(BF16),
                            (((1,), (1,)), ((), ())),
                            preferred_element_type=F32) * (HD ** -0.5)
        s = jnp.where(col < NV, s, -1e30)
        p = jax.nn.softmax(s, axis=-1)
        outs.append(_bdot(p, v))
    o = jnp.concatenate(outs, axis=1)
    out_ref[0] = x + _bdot(o, pw_ref[...]) + pb_ref[...]


def _attn_block(t, p, y=None, valid=None):
    fuse = y is not None
    ins = [t]
    in_specs = [pl.BlockSpec((1, NP, D), lambda b: (b, 0, 0))]
    if fuse:
        ins += [y.reshape(B, NP, D), valid.reshape(B, 1, NP)]
        in_specs += [pl.BlockSpec((1, NP, D), lambda b: (b, 0, 0)),
                     pl.BlockSpec((1, 1, NP), lambda b: (b, 0, 0))]
    ins += [p['ln1_g'], p['ln1_b'], p['qkv_w'], p['qkv_b'],
            p['proj_w'], p['proj_b']]
    in_specs += [
        pl.BlockSpec((D,), lambda b: (0,)),
        pl.BlockSpec((D,), lambda b: (0,)),
        pl.BlockSpec((D, 3 * D), lambda b: (0, 0)),
        pl.BlockSpec((3 * D,), lambda b: (0,)),
        pl.BlockSpec((D, D), lambda b: (0, 0)),
        pl.BlockSpec((D,), lambda b: (0,)),
    ]
    return pl.pallas_call(
        functools.partial(_attn_krn, fuse),
        grid=(B,),
        in_specs=in_specs,
        out_specs=pl.BlockSpec((1, NP, D), lambda b: (b, 0, 0)),
        out_shape=jax.ShapeDtypeStruct((B, NP, D), F32),
    )(*ins)


def _mlp_krn(t_ref, g_ref, b_ref, w1_ref, b1_ref, w2_ref, b2_ref,
             out_ref, xln_s):
    j = pl.program_id(0)

    @pl.when(j == 0)
    def _():
        x = t_ref[...]
        xln_s[...] = _ln(x, g_ref[...], b_ref[...])
        out_ref[...] = x + b2_ref[...]

    h = jax.nn.gelu(_bdot(xln_s[...], w1_ref[...]) + b1_ref[0])
    out_ref[...] += _bdot(h, w2_ref[...])


def _mlp_block(t, p):
    nj = HID // HC
    tf = t.reshape(T, D)
    out = pl.pallas_call(
        _mlp_krn,
        grid=(nj,),
        in_specs=[
            pl.BlockSpec((T, D), lambda j: (0, 0)),
            pl.BlockSpec((D,), lambda j: (0,)),
            pl.BlockSpec((D,), lambda j: (0,)),
            pl.BlockSpec((D, HC), lambda j: (0, j)),
            pl.BlockSpec((1, 1, HC), lambda j: (j, 0, 0)),
            pl.BlockSpec((HC, D), lambda j: (j, 0)),
            pl.BlockSpec((D,), lambda j: (0,)),
        ],
        out_specs=pl.BlockSpec((T, D), lambda j: (0, 0)),
        out_shape=jax.ShapeDtypeStruct((T, D), F32),
        scratch_shapes=[pltpu.VMEM((T, D), F32)],
    )(tf, p['ln2_g'], p['ln2_b'], p['fc1_w'],
      p['fc1_b'].reshape(nj, 1, HC), p['fc2_w'], p['fc2_b'])
    return out.reshape(B, NP, D)


def _route_krn(t_ref, g_ref, b_ref, gw_ref, xln_ref, src_ref, comb_ref,
               val_ref):
    x = t_ref[...]
    xln = _ln(x, g_ref[...], b_ref[...])
    xln_ref[...] = xln
    logits = lax.dot_general(xln, gw_ref[...], (((1,), (0,)), ((), ())),
                             preferred_element_type=F32,
                             precision=lax.Precision.HIGHEST)
    lane = lax.broadcasted_iota(jnp.int32, (T, E), 1)
    mx = jnp.max(logits, axis=1, keepdims=True)
    e = jnp.min(jnp.where(logits >= mx, lane, E), axis=1)
    row = lax.broadcasted_iota(jnp.int32, (T, E), 0)
    tokmask = (row % NP) < NV
    oh = ((lane == e[:, None]) & tokmask).astype(BF16)
    tri = (lax.broadcasted_iota(jnp.int32, (T, T), 0)
           >= lax.broadcasted_iota(jnp.int32, (T, T), 1)).astype(BF16)
    cum = lax.dot_general(tri, oh, (((1,), (0,)), ((), ())),
                          preferred_element_type=F32)
    pos = jnp.sum(jnp.where(lane == e[:, None], cum - 1.0, 0.0),
                  axis=1).astype(jnp.int32)
    valid = (pos < C) & tokmask[:, 0]
    slot = jnp.where(valid, e * CP + pos, TRASH)
    comb = e * CP + jnp.clip(pos, 0, C - 1)
    sm = (slot[:, None]
          == lax.broadcasted_iota(jnp.int32, (T, EC), 1)).astype(BF16)
    tok_hi = (lax.broadcasted_iota(jnp.int32, (1, T), 1) >> 8).astype(BF16)
    tok_lo = (lax.broadcasted_iota(jnp.int32, (1, T), 1) & 255).astype(BF16)
    srcf = (lax.dot_general(tok_hi, sm, (((1,), (0,)), ((), ())),
                            preferred_element_type=F32) * 256.0
            + lax.dot_general(tok_lo, sm, (((1,), (0,)), ((), ())),
                              preferred_element_type=F32))
    src_ref[...] = srcf.reshape(EC).astype(jnp.int32)
    comb_ref[...] = comb.astype(jnp.int32)
    val_ref[...] = valid.astype(F32)


def _route(t, p):
    tf = t.reshape(T, D)
    return pl.pallas_call(
        _route_krn,
        grid=(1,),
        in_specs=[
            pl.BlockSpec((T, D), lambda i: (0, 0)),
            pl.BlockSpec((D,), lambda i: (0,)),
            pl.BlockSpec((D,), lambda i: (0,)),
            pl.BlockSpec((D, E), lambda i: (0, 0)),
        ],
        out_specs=[
            pl.BlockSpec((T, D), lambda i: (0, 0)),
            pl.BlockSpec((EC,), lambda i: (0,)),
            pl.BlockSpec((T,), lambda i: (0,)),
            pl.BlockSpec((T,), lambda i: (0,)),
        ],
        out_shape=[
            jax.ShapeDtypeStruct((T, D), F32),
            jax.ShapeDtypeStruct((EC,), jnp.int32),
            jax.ShapeDtypeStruct((T,), jnp.int32),
            jax.ShapeDtypeStruct((T,), F32),
        ],
    )(tf, p['ln2_g'], p['ln2_b'], p['gate_w'])


def _expert_krn(x_ref, w1_ref, b1_ref, w2_ref, b2_ref, out_ref):
    j = pl.program_id(1)

    @pl.when(j == 0)
    def _():
        out_ref[...] = jnp.broadcast_to(b2_ref[0], (CP, D))

    h = jax.nn.gelu(_bdot(x_ref[...], w1_ref[0]) + b1_ref[0, 0])
    out_ref[...] += _bdot(h, w2_ref[0])


def _experts(buf, p):
    nj = HID // HC
    return pl.pallas_call(
        _expert_krn,
        grid=(E, nj),
        in_specs=[
            pl.BlockSpec((CP, D), lambda e, j: (e, 0)),
            pl.BlockSpec((1, D, HC), lambda e, j: (e, 0, j)),
            pl.BlockSpec((1, 1, 1, HC), lambda e, j: (e, j, 0, 0)),
            pl.BlockSpec((1, HC, D), lambda e, j: (e, j, 0)),
            pl.BlockSpec((1, 1, D), lambda e, j: (e, 0, 0)),
        ],
        out_specs=pl.BlockSpec((CP, D), lambda e, j: (e, 0)),
        out_shape=jax.ShapeDtypeStruct((EC, D), F32),
    )(buf, p['w1'], p['b1'].reshape(E, nj, 1, HC), p['w2'],
      p['b2'].reshape(E, 1, D))


SRCN = 2064
ROWS_D = EC // 32
ROWS_C = 64
NW_C = T // ROWS_C


def _sc_mesh():
    return plsc.VectorSubcoreMesh(core_axis_name="c", subcore_axis_name="s")


def _sc_disp_krn(src_hbm, x_hbm, buf_hbm, idx_v, rows_v, sem):
    wid = lax.axis_index("s") * 2 + lax.axis_index("c")
    base = wid * ROWS_D
    pltpu.sync_copy(src_hbm.at[pl.ds(base, ROWS_D)], idx_v)
    pltpu.async_copy(x_hbm.at[idx_v], rows_v, sem).wait()
    pltpu.sync_copy(rows_v, buf_hbm.at[pl.ds(base, ROWS_D)])


def _sc_dispatch(src, xln):
    return pl.kernel(
        _sc_disp_krn,
        out_type=jax.ShapeDtypeStruct((EC, D), F32),
        mesh=_sc_mesh(),
        compiler_params=pltpu.CompilerParams(needs_layout_passes=False),
        scratch_types=[
            pltpu.VMEM((ROWS_D,), jnp.int32),
            pltpu.VMEM((ROWS_D, D), F32),
            pltpu.SemaphoreType.DMA,
        ],
    )(src, xln)


def _sc_comb_krn(comb_hbm, out_hbm, y_hbm, idx_v, rows_v, sem):
    wid = lax.axis_index("s") * 2 + lax.axis_index("c")

    @pl.when(wid < NW_C)
    def _():
        base = wid * ROWS_C
        pltpu.sync_copy(comb_hbm.at[pl.ds(base, ROWS_C)], idx_v)
        pltpu.async_copy(out_hbm.at[idx_v], rows_v, sem).wait()
        pltpu.sync_copy(rows_v, y_hbm.at[pl.ds(base, ROWS_C)])


def _sc_combine(comb, out):
    return pl.kernel(
        _sc_comb_krn,
        out_type=jax.ShapeDtypeStruct((T, D), F32),
        mesh=_sc_mesh(),
        compiler_params=pltpu.CompilerParams(needs_layout_passes=False),
        scratch_types=[
            pltpu.VMEM((ROWS_C,), jnp.int32),
            pltpu.VMEM((ROWS_C, D), F32),
            pltpu.SemaphoreType.DMA,
        ],
    )(comb, out)


def _head_krn(t_ref, y_ref, val_ref, g_ref, b_ref, hw_ref, hb_ref,
              out_ref):
    x = t_ref[:, 0, :] + y_ref[:, 0, :] * val_ref[:, 0, 0:1]
    xln = _ln(x, g_ref[...], b_ref[...])
    out_ref[...] = _bdot(xln, hw_ref[...]) + hb_ref[...]


def _head(t, y, valid, params):
    return pl.pallas_call(
        _head_krn,
        grid=(1,),
        in_specs=[
            pl.BlockSpec((B, 8, D), lambda i: (0, 0, 0)),
            pl.BlockSpec((B, 8, D), lambda i: (0, 0, 0)),
            pl.BlockSpec((B, 1, NP), lambda i: (0, 0, 0)),
            pl.BlockSpec((D,), lambda i: (0,)),
            pl.BlockSpec((D,), lambda i: (0,)),
            pl.BlockSpec((D, 1000), lambda i: (0, 0)),
            pl.BlockSpec((1000,), lambda i: (0,)),
        ],
        out_specs=pl.BlockSpec((B, 1000), lambda i: (0, 0)),
        out_shape=jax.ShapeDtypeStruct((B, 1000), F32),
    )(t, y.reshape(B, NP, D), valid.reshape(B, 1, NP),
      params['ln_g'], params['ln_b'], params['head_w'], params['head_b'])


def kernel(x, params):
    GP, PATCH = 14, 16
    patches = x.reshape(B, 3, GP, PATCH, GP, PATCH)
    patches = patches.transpose(0, 2, 4, 1, 3, 5).reshape(
        B, GP * GP, 3 * PATCH * PATCH)
    pos = params['pos'][0]
    pos_pad = jnp.concatenate([pos, jnp.zeros((NP - NV, D), F32)], axis=0)
    cls_vec = params['cls'].reshape(1, D)
    t = _embed(patches, params['patch_w'], params['patch_b'], cls_vec,
               pos_pad)

    y, valid = None, None
    for p in params['blocks']:
        t = _attn_block(t, p, y, valid)
        y, valid = None, None
        if 'gate_w' in p:
            xln, src, comb, valid = _route(t, p)
            buf = _sc_dispatch(src, xln)
            out = _experts(buf, p)
            y = _sc_combine(comb, out)
        else:
            t = _mlp_block(t, p)
    return _head(t, y, valid, params)


# speedup vs baseline: 1.0416x; 1.0416x over previous
import functools
import math

import jax
import jax.numpy as jnp
from jax import lax
from jax.experimental import pallas as pl
from jax.experimental.pallas import tpu as pltpu
from jax.experimental.pallas import tpu_sc as plsc

D = 768
H = 12
HD = 64
E = 8
HID = 3072
B = 8
NV = 197
NP = 200
T = B * NP
TV = B * NV
C = int(math.ceil(TV * 1.0 / E * 1.25))
CP = 256
EC = E * CP
TRASH = EC
HC = 768
F32 = jnp.float32
BF16 = jnp.bfloat16


def _ln(x, g, b):
    m = jnp.mean(x, axis=-1, keepdims=True)
    v = jnp.mean((x - m) ** 2, axis=-1, keepdims=True)
    return (x - m) / jnp.sqrt(v + 1e-6) * g + b


def _bdot(a, b):
    return lax.dot_general(a.astype(BF16), b.astype(BF16),
                           (((1,), (0,)), ((), ())),
                           preferred_element_type=F32)


def _embed_krn(p_ref, pw_ref, pb_ref, cls_ref, pos_ref, out_ref):
    body = _bdot(p_ref[0], pw_ref[...]) + pb_ref[...]
    full = jnp.concatenate(
        [cls_ref[...], body, jnp.zeros((NP - NV, D), F32)], axis=0)
    out_ref[0] = full + pos_ref[...]


def _embed(patches, pw, pb, cls_vec, pos_pad):
    return pl.pallas_call(
        _embed_krn,
        grid=(B,),
        in_specs=[
            pl.BlockSpec((1, NV - 1, D), lambda b: (b, 0, 0)),
            pl.BlockSpec((D, D), lambda b: (0, 0)),
            pl.BlockSpec((D,), lambda b: (0,)),
            pl.BlockSpec((1, D), lambda b: (0, 0)),
            pl.BlockSpec((NP, D), lambda b: (0, 0)),
        ],
        out_specs=pl.BlockSpec((1, NP, D), lambda b: (b, 0, 0)),
        out_shape=jax.ShapeDtypeStruct((B, NP, D), F32),
    )(patches, pw, pb, cls_vec, pos_pad)


def _attn_krn(fuse_y, *refs):
    if fuse_y:
        (t_ref, sm_ref, eo_ref, g_ref, b_ref, qw_ref, qb_ref,
         pw_ref, pb_ref, out_ref) = refs
        x = t_ref[0] + lax.dot_general(sm_ref[0], eo_ref[...],
                                       (((1,), (0,)), ((), ())),
                                       preferred_element_type=F32)
    else:
        (t_ref, g_ref, b_ref, qw_ref, qb_ref, pw_ref, pb_ref,
         out_ref) = refs
        x = t_ref[0]
    xln = _ln(x, g_ref[...], b_ref[...])
    qkv = _bdot(xln, qw_ref[...]) + qb_ref[...]
    col = lax.broadcasted_iota(jnp.int32, (NP, NP), 1)
    outs = []
    for h in range(H):
        q = qkv[:, h * HD:(h + 1) * HD]
        k = qkv[:, D + h * HD:D + (h + 1) * HD]
        v = qkv[:, 2 * D + h * HD:2 * D + (h + 1) * HD]
        s = lax.dot_general(q.astype(BF16), k.astype(BF16),
                            (((1,), (1,)), ((), ())),
                            preferred_element_type=F32) * (HD ** -0.5)
        s = jnp.where(col < NV, s, -1e30)
        p = jax.nn.softmax(s, axis=-1)
        outs.append(_bdot(p, v))
    o = jnp.concatenate(outs, axis=1)
    out_ref[0] = x + _bdot(o, pw_ref[...]) + pb_ref[...]


def _attn_block(t, p, sm=None, eo=None):
    fuse = sm is not None
    ins = [t]
    in_specs = [pl.BlockSpec((1, NP, D), lambda b: (b, 0, 0))]
    if fuse:
        ins += [sm.reshape(B, NP, EC), eo]
        in_specs += [pl.BlockSpec((1, NP, EC), lambda b: (b, 0, 0)),
                     pl.BlockSpec((EC, D), lambda b: (0, 0))]
    ins += [p['ln1_g'], p['ln1_b'], p['qkv_w'], p['qkv_b'],
            p['proj_w'], p['proj_b']]
    in_specs += [
        pl.BlockSpec((D,), lambda b: (0,)),
        pl.BlockSpec((D,), lambda b: (0,)),
        pl.BlockSpec((D, 3 * D), lambda b: (0, 0)),
        pl.BlockSpec((3 * D,), lambda b: (0,)),
        pl.BlockSpec((D, D), lambda b: (0, 0)),
        pl.BlockSpec((D,), lambda b: (0,)),
    ]
    return pl.pallas_call(
        functools.partial(_attn_krn, fuse),
        grid=(B,),
        in_specs=in_specs,
        out_specs=pl.BlockSpec((1, NP, D), lambda b: (b, 0, 0)),
        out_shape=jax.ShapeDtypeStruct((B, NP, D), F32),
    )(*ins)


def _mlp_krn(t_ref, g_ref, b_ref, w1_ref, b1_ref, w2_ref, b2_ref,
             out_ref, xln_s):
    j = pl.program_id(0)

    @pl.when(j == 0)
    def _():
        x = t_ref[...]
        xln_s[...] = _ln(x, g_ref[...], b_ref[...])
        out_ref[...] = x + b2_ref[...]

    h = jax.nn.gelu(_bdot(xln_s[...], w1_ref[...]) + b1_ref[0])
    out_ref[...] += _bdot(h, w2_ref[...])


def _mlp_block(t, p):
    nj = HID // HC
    tf = t.reshape(T, D)
    out = pl.pallas_call(
        _mlp_krn,
        grid=(nj,),
        in_specs=[
            pl.BlockSpec((T, D), lambda j: (0, 0)),
            pl.BlockSpec((D,), lambda j: (0,)),
            pl.BlockSpec((D,), lambda j: (0,)),
            pl.BlockSpec((D, HC), lambda j: (0, j)),
            pl.BlockSpec((1, 1, HC), lambda j: (j, 0, 0)),
            pl.BlockSpec((HC, D), lambda j: (j, 0)),
            pl.BlockSpec((D,), lambda j: (0,)),
        ],
        out_specs=pl.BlockSpec((T, D), lambda j: (0, 0)),
        out_shape=jax.ShapeDtypeStruct((T, D), F32),
        scratch_shapes=[pltpu.VMEM((T, D), F32)],
    )(tf, p['ln2_g'], p['ln2_b'], p['fc1_w'],
      p['fc1_b'].reshape(nj, 1, HC), p['fc2_w'], p['fc2_b'])
    return out.reshape(B, NP, D)


def _route_krn(t_ref, g_ref, b_ref, gw_ref, xln_ref, src_ref, sm_ref):
    x = t_ref[...]
    xln = _ln(x, g_ref[...], b_ref[...])
    xln_ref[...] = xln
    logits = lax.dot_general(xln, gw_ref[...], (((1,), (0,)), ((), ())),
                             preferred_element_type=F32,
                             precision=lax.Precision.HIGHEST)
    lane = lax.broadcasted_iota(jnp.int32, (T, E), 1)
    mx = jnp.max(logits, axis=1, keepdims=True)
    e = jnp.min(jnp.where(logits >= mx, lane, E), axis=1)
    row = lax.broadcasted_iota(jnp.int32, (T, E), 0)
    tokmask = (row % NP) < NV
    oh = ((lane == e[:, None]) & tokmask).astype(BF16)
    tri = (lax.broadcasted_iota(jnp.int32, (T, T), 0)
           >= lax.broadcasted_iota(jnp.int32, (T, T), 1)).astype(BF16)
    cum = lax.dot_general(tri, oh, (((1,), (0,)), ((), ())),
                          preferred_element_type=F32)
    pos = jnp.sum(jnp.where(lane == e[:, None], cum - 1.0, 0.0),
                  axis=1).astype(jnp.int32)
    valid = (pos < C) & tokmask[:, 0]
    slot = jnp.where(valid, e * CP + pos, TRASH)
    sm = (slot[:, None]
          == lax.broadcasted_iota(jnp.int32, (T, EC), 1)).astype(BF16)
    tok_hi = (lax.broadcasted_iota(jnp.int32, (1, T), 1) >> 8).astype(BF16)
    tok_lo = (lax.broadcasted_iota(jnp.int32, (1, T), 1) & 255).astype(BF16)
    srcf = (lax.dot_general(tok_hi, sm, (((1,), (0,)), ((), ())),
                            preferred_element_type=F32) * 256.0
            + lax.dot_general(tok_lo, sm, (((1,), (0,)), ((), ())),
                              preferred_element_type=F32))
    src_ref[...] = srcf.reshape(EC).astype(jnp.int32)
    sm_ref[...] = sm


def _route(t, p):
    tf = t.reshape(T, D)
    return pl.pallas_call(
        _route_krn,
        grid=(1,),
        in_specs=[
            pl.BlockSpec((T, D), lambda i: (0, 0)),
            pl.BlockSpec((D,), lambda i: (0,)),
            pl.BlockSpec((D,), lambda i: (0,)),
            pl.BlockSpec((D, E), lambda i: (0, 0)),
        ],
        out_specs=[
            pl.BlockSpec((T, D), lambda i: (0, 0)),
            pl.BlockSpec((EC,), lambda i: (0,)),
            pl.BlockSpec((T, EC), lambda i: (0, 0)),
        ],
        out_shape=[
            jax.ShapeDtypeStruct((T, D), F32),
            jax.ShapeDtypeStruct((EC,), jnp.int32),
            jax.ShapeDtypeStruct((T, EC), BF16),
        ],
    )(tf, p['ln2_g'], p['ln2_b'], p['gate_w'])


def _expert_krn(x_ref, w1_ref, b1_ref, w2_ref, b2_ref, out_ref, acc_s):
    j = pl.program_id(1)

    @pl.when(j == 0)
    def _():
        acc_s[...] = jnp.broadcast_to(b2_ref[0], (CP, D))

    h = jax.nn.gelu(_bdot(x_ref[...], w1_ref[0]) + b1_ref[0, 0])
    acc_s[...] += _bdot(h, w2_ref[0])

    @pl.when(j == HID // HC - 1)
    def _():
        out_ref[...] = acc_s[...].astype(BF16)


def _experts(buf, p):
    nj = HID // HC
    return pl.pallas_call(
        _expert_krn,
        grid=(E, nj),
        in_specs=[
            pl.BlockSpec((CP, D), lambda e, j: (e, 0)),
            pl.BlockSpec((1, D, HC), lambda e, j: (e, 0, j)),
            pl.BlockSpec((1, 1, 1, HC), lambda e, j: (e, j, 0, 0)),
            pl.BlockSpec((1, HC, D), lambda e, j: (e, j, 0)),
            pl.BlockSpec((1, 1, D), lambda e, j: (e, 0, 0)),
        ],
        out_specs=pl.BlockSpec((CP, D), lambda e, j: (e, 0)),
        out_shape=jax.ShapeDtypeStruct((EC, D), BF16),
        scratch_shapes=[pltpu.VMEM((CP, D), F32)],
    )(buf, p['w1'], p['b1'].reshape(E, nj, 1, HC), p['w2'],
      p['b2'].reshape(E, 1, D))


SRCN = 2064
ROWS_D = EC // 32
ROWS_C = 64
NW_C = T // ROWS_C


def _sc_mesh():
    return plsc.VectorSubcoreMesh(core_axis_name="c", subcore_axis_name="s")


def _sc_disp_krn(src_hbm, x_hbm, buf_hbm, idx_v, rows_v, sem):
    wid = lax.axis_index("s") * 2 + lax.axis_index("c")
    base = wid * ROWS_D
    pltpu.sync_copy(src_hbm.at[pl.ds(base, ROWS_D)], idx_v)
    pltpu.async_copy(x_hbm.at[idx_v], rows_v, sem).wait()
    pltpu.sync_copy(rows_v, buf_hbm.at[pl.ds(base, ROWS_D)])


def _sc_dispatch(src, xln):
    return pl.kernel(
        _sc_disp_krn,
        out_type=jax.ShapeDtypeStruct((EC, D), F32),
        mesh=_sc_mesh(),
        compiler_params=pltpu.CompilerParams(needs_layout_passes=False),
        scratch_types=[
            pltpu.VMEM((ROWS_D,), jnp.int32),
            pltpu.VMEM((ROWS_D, D), F32),
            pltpu.SemaphoreType.DMA,
        ],
    )(src, xln)


def _head_krn(t_ref, sm_ref, eo_ref, g_ref, b_ref, hw_ref, hb_ref,
              out_ref):
    y0 = lax.dot_general(sm_ref[0, 0:1, :], eo_ref[...],
                         (((1,), (0,)), ((), ())),
                         preferred_element_type=F32)
    x = t_ref[0, 0:1, :] + y0
    xln = _ln(x, g_ref[...], b_ref[...])
    out_ref[0] = _bdot(xln, hw_ref[...]) + hb_ref[...]


def _head(t, sm, eo, params):
    out = pl.pallas_call(
        _head_krn,
        grid=(B,),
        in_specs=[
            pl.BlockSpec((1, 8, D), lambda b: (b, 0, 0)),
            pl.BlockSpec((1, 8, EC), lambda b: (b, 0, 0)),
            pl.BlockSpec((EC, D), lambda b: (0, 0)),
            pl.BlockSpec((D,), lambda b: (0,)),
            pl.BlockSpec((D,), lambda b: (0,)),
            pl.BlockSpec((D, 1000), lambda b: (0, 0)),
            pl.BlockSpec((1000,), lambda b: (0,)),
        ],
        out_specs=pl.BlockSpec((1, 1, 1000), lambda b: (b, 0, 0)),
        out_shape=jax.ShapeDtypeStruct((B, 1, 1000), F32),
    )(t, sm.reshape(B, NP, EC), eo,
      params['ln_g'], params['ln_b'], params['head_w'], params['head_b'])
    return out.reshape(B, 1000)


def kernel(x, params):
    GP, PATCH = 14, 16
    patches = x.reshape(B, 3, GP, PATCH, GP, PATCH)
    patches = patches.transpose(0, 2, 4, 1, 3, 5).reshape(
        B, GP * GP, 3 * PATCH * PATCH)
    pos = params['pos'][0]
    pos_pad = jnp.concatenate([pos, jnp.zeros((NP - NV, D), F32)], axis=0)
    cls_vec = params['cls'].reshape(1, D)
    t = _embed(patches, params['patch_w'], params['patch_b'], cls_vec,
               pos_pad)

    sm, eo = None, None
    for p in params['blocks']:
        t = _attn_block(t, p, sm, eo)
        sm, eo = None, None
        if 'gate_w' in p:
            xln, src, sm = _route(t, p)
            buf = _sc_dispatch(src, xln)
            eo = _experts(buf, p)
        else:
            t = _mlp_block(t, p)
    return _head(t, sm, eo, params)


# speedup vs baseline: 1.0852x; 1.0419x over previous
import functools
import math

import jax
import jax.numpy as jnp
from jax import lax
from jax.experimental import pallas as pl
from jax.experimental.pallas import tpu as pltpu
from jax.experimental.pallas import tpu_sc as plsc

D = 768
H = 12
HD = 64
E = 8
HID = 3072
B = 8
NV = 197
NP = 200
T = B * NP
TV = B * NV
C = int(math.ceil(TV * 1.0 / E * 1.25))
CP = 256
EC = E * CP
TRASH = EC
HC = 768
HCE = 1536
F32 = jnp.float32
BF16 = jnp.bfloat16


def _ln(x, g, b):
    m = jnp.mean(x, axis=-1, keepdims=True)
    v = jnp.mean((x - m) ** 2, axis=-1, keepdims=True)
    return (x - m) / jnp.sqrt(v + 1e-6) * g + b


def _bdot(a, b):
    return lax.dot_general(a.astype(BF16), b.astype(BF16),
                           (((1,), (0,)), ((), ())),
                           preferred_element_type=F32)


def _embed_krn(p_ref, pw_ref, pb_ref, cls_ref, pos_ref, out_ref):
    body = _bdot(p_ref[0], pw_ref[...]) + pb_ref[...]
    full = jnp.concatenate(
        [cls_ref[...], body, jnp.zeros((NP - NV, D), F32)], axis=0)
    out_ref[0] = full + pos_ref[...]


def _embed(patches, pw, pb, cls_vec, pos_pad):
    return pl.pallas_call(
        _embed_krn,
        grid=(B,),
        in_specs=[
            pl.BlockSpec((1, NV - 1, D), lambda b: (b, 0, 0)),
            pl.BlockSpec((D, D), lambda b: (0, 0)),
            pl.BlockSpec((D,), lambda b: (0,)),
            pl.BlockSpec((1, D), lambda b: (0, 0)),
            pl.BlockSpec((NP, D), lambda b: (0, 0)),
        ],
        out_specs=pl.BlockSpec((1, NP, D), lambda b: (b, 0, 0)),
        out_shape=jax.ShapeDtypeStruct((B, NP, D), F32),
    )(patches, pw, pb, cls_vec, pos_pad)


def _attn_krn(fuse_y, *refs):
    if fuse_y:
        (t_ref, sm_ref, eo_ref, g_ref, b_ref, qw_ref, qb_ref,
         pw_ref, pb_ref, out_ref) = refs
        x = t_ref[0] + lax.dot_general(sm_ref[0], eo_ref[...],
                                       (((1,), (0,)), ((), ())),
                                       preferred_element_type=F32)
    else:
        (t_ref, g_ref, b_ref, qw_ref, qb_ref, pw_ref, pb_ref,
         out_ref) = refs
        x = t_ref[0]
    xln = _ln(x, g_ref[...], b_ref[...])
    qkv = _bdot(xln, qw_ref[...]) + qb_ref[...]
    col = lax.broadcasted_iota(jnp.int32, (NP, NP), 1)
    outs = []
    for h in range(H):
        q = qkv[:, h * HD:(h + 1) * HD]
        k = qkv[:, D + h * HD:D + (h + 1) * HD]
        v = qkv[:, 2 * D + h * HD:2 * D + (h + 1) * HD]
        s = lax.dot_general(q.astype(BF16), k.astype(BF16),
                            (((1,), (1,)), ((), ())),
                            preferred_element_type=F32) * (HD ** -0.5)
        s = jnp.where(col < NV, s, -1e30)
        p = jax.nn.softmax(s, axis=-1)
        outs.append(_bdot(p, v))
    o = jnp.concatenate(outs, axis=1)
    out_ref[0] = x + _bdot(o, pw_ref[...]) + pb_ref[...]


def _attn_block(t, p, sm=None, eo=None):
    fuse = sm is not None
    ins = [t]
    in_specs = [pl.BlockSpec((1, NP, D), lambda b: (b, 0, 0))]
    if fuse:
        ins += [sm.reshape(B, NP, EC), eo]
        in_specs += [pl.BlockSpec((1, NP, EC), lambda b: (b, 0, 0)),
                     pl.BlockSpec((EC, D), lambda b: (0, 0))]
    ins += [p['ln1_g'], p['ln1_b'], p['qkv_w'], p['qkv_b'],
            p['proj_w'], p['proj_b']]
    in_specs += [
        pl.BlockSpec((D,), lambda b: (0,)),
        pl.BlockSpec((D,), lambda b: (0,)),
        pl.BlockSpec((D, 3 * D), lambda b: (0, 0)),
        pl.BlockSpec((3 * D,), lambda b: (0,)),
        pl.BlockSpec((D, D), lambda b: (0, 0)),
        pl.BlockSpec((D,), lambda b: (0,)),
    ]
    return pl.pallas_call(
        functools.partial(_attn_krn, fuse),
        grid=(B,),
        in_specs=in_specs,
        out_specs=pl.BlockSpec((1, NP, D), lambda b: (b, 0, 0)),
        out_shape=jax.ShapeDtypeStruct((B, NP, D), F32),
    )(*ins)


def _mlp_krn(t_ref, g_ref, b_ref, w1_ref, b1_ref, w2_ref, b2_ref,
             out_ref, xln_s):
    j = pl.program_id(0)

    @pl.when(j == 0)
    def _():
        x = t_ref[...]
        xln_s[...] = _ln(x, g_ref[...], b_ref[...])
        out_ref[...] = x + b2_ref[...]

    h = jax.nn.gelu(_bdot(xln_s[...], w1_ref[...]) + b1_ref[0])
    out_ref[...] += _bdot(h, w2_ref[...])


def _mlp_block(t, p):
    nj = HID // HC
    tf = t.reshape(T, D)
    out = pl.pallas_call(
        _mlp_krn,
        grid=(nj,),
        in_specs=[
            pl.BlockSpec((T, D), lambda j: (0, 0)),
            pl.BlockSpec((D,), lambda j: (0,)),
            pl.BlockSpec((D,), lambda j: (0,)),
            pl.BlockSpec((D, HC), lambda j: (0, j)),
            pl.BlockSpec((1, 1, HC), lambda j: (j, 0, 0)),
            pl.BlockSpec((HC, D), lambda j: (j, 0)),
            pl.BlockSpec((D,), lambda j: (0,)),
        ],
        out_specs=pl.BlockSpec((T, D), lambda j: (0, 0)),
        out_shape=jax.ShapeDtypeStruct((T, D), F32),
        scratch_shapes=[pltpu.VMEM((T, D), F32)],
    )(tf, p['ln2_g'], p['ln2_b'], p['fc1_w'],
      p['fc1_b'].reshape(nj, 1, HC), p['fc2_w'], p['fc2_b'])
    return out.reshape(B, NP, D)


def _route_krn(t_ref, g_ref, b_ref, gw_ref, xln_ref, src_ref, sm_ref):
    x = t_ref[...]
    xln = _ln(x, g_ref[...], b_ref[...])
    xln_ref[...] = xln
    logits = lax.dot_general(xln, gw_ref[...], (((1,), (0,)), ((), ())),
                             preferred_element_type=F32,
                             precision=lax.Precision.HIGHEST)
    lane = lax.broadcasted_iota(jnp.int32, (T, E), 1)
    mx = jnp.max(logits, axis=1, keepdims=True)
    e = jnp.min(jnp.where(logits >= mx, lane, E), axis=1)
    row = lax.broadcasted_iota(jnp.int32, (T, E), 0)
    tokmask = (row % NP) < NV
    oh = ((lane == e[:, None]) & tokmask).astype(BF16)
    tri = (lax.broadcasted_iota(jnp.int32, (T, T), 0)
           >= lax.broadcasted_iota(jnp.int32, (T, T), 1)).astype(BF16)
    cum = lax.dot_general(tri, oh, (((1,), (0,)), ((), ())),
                          preferred_element_type=F32)
    pos = jnp.sum(jnp.where(lane == e[:, None], cum - 1.0, 0.0),
                  axis=1).astype(jnp.int32)
    valid = (pos < C) & tokmask[:, 0]
    slot = jnp.where(valid, e * CP + pos, TRASH)
    sm = (slot[:, None]
          == lax.broadcasted_iota(jnp.int32, (T, EC), 1)).astype(BF16)
    tok_hi = (lax.broadcasted_iota(jnp.int32, (1, T), 1) >> 8).astype(BF16)
    tok_lo = (lax.broadcasted_iota(jnp.int32, (1, T), 1) & 255).astype(BF16)
    srcf = (lax.dot_general(tok_hi, sm, (((1,), (0,)), ((), ())),
                            preferred_element_type=F32) * 256.0
            + lax.dot_general(tok_lo, sm, (((1,), (0,)), ((), ())),
                              preferred_element_type=F32))
    src_ref[...] = srcf.reshape(EC).astype(jnp.int32)
    sm_ref[...] = sm


def _route(t, p):
    tf = t.reshape(T, D)
    return pl.pallas_call(
        _route_krn,
        grid=(1,),
        in_specs=[
            pl.BlockSpec((T, D), lambda i: (0, 0)),
            pl.BlockSpec((D,), lambda i: (0,)),
            pl.BlockSpec((D,), lambda i: (0,)),
            pl.BlockSpec((D, E), lambda i: (0, 0)),
        ],
        out_specs=[
            pl.BlockSpec((T, D), lambda i: (0, 0)),
            pl.BlockSpec((EC,), lambda i: (0,)),
            pl.BlockSpec((T, EC), lambda i: (0, 0)),
        ],
        out_shape=[
            jax.ShapeDtypeStruct((T, D), F32),
            jax.ShapeDtypeStruct((EC,), jnp.int32),
            jax.ShapeDtypeStruct((T, EC), BF16),
        ],
    )(tf, p['ln2_g'], p['ln2_b'], p['gate_w'])


def _expert_krn(x_ref, w1_ref, b1_ref, w2_ref, b2_ref, out_ref, acc_s):
    j = pl.program_id(1)

    @pl.when(j == 0)
    def _():
        acc_s[...] = jnp.broadcast_to(b2_ref[0], (CP, D))

    h = jax.nn.gelu(_bdot(x_ref[...], w1_ref[0]) + b1_ref[0, 0])
    acc_s[...] += _bdot(h, w2_ref[0])

    @pl.when(j == HID // HCE - 1)
    def _():
        out_ref[...] = acc_s[...].astype(BF16)


def _experts(buf, p):
    nj = HID // HCE
    return pl.pallas_call(
        _expert_krn,
        grid=(E, nj),
        in_specs=[
            pl.BlockSpec((CP, D), lambda e, j: (e, 0)),
            pl.BlockSpec((1, D, HCE), lambda e, j: (e, 0, j)),
            pl.BlockSpec((1, 1, 1, HCE), lambda e, j: (e, j, 0, 0)),
            pl.BlockSpec((1, HCE, D), lambda e, j: (e, j, 0)),
            pl.BlockSpec((1, 1, D), lambda e, j: (e, 0, 0)),
        ],
        out_specs=pl.BlockSpec((CP, D), lambda e, j: (e, 0)),
        out_shape=jax.ShapeDtypeStruct((EC, D), BF16),
        scratch_shapes=[pltpu.VMEM((CP, D), F32)],
    )(buf, p['w1'], p['b1'].reshape(E, nj, 1, HCE), p['w2'],
      p['b2'].reshape(E, 1, D))


SRCN = 2064
ROWS_D = EC // 32
ROWS_C = 64
NW_C = T // ROWS_C


def _sc_mesh():
    return plsc.VectorSubcoreMesh(core_axis_name="c", subcore_axis_name="s")


def _sc_disp_krn(src_hbm, x_hbm, buf_hbm, idx_v, rows_v, sem):
    wid = lax.axis_index("s") * 2 + lax.axis_index("c")
    base = wid * ROWS_D
    pltpu.sync_copy(src_hbm.at[pl.ds(base, ROWS_D)], idx_v)
    pltpu.async_copy(x_hbm.at[idx_v], rows_v, sem).wait()
    pltpu.sync_copy(rows_v, buf_hbm.at[pl.ds(base, ROWS_D)])


def _sc_dispatch(src, xln):
    return pl.kernel(
        _sc_disp_krn,
        out_type=jax.ShapeDtypeStruct((EC, D), F32),
        mesh=_sc_mesh(),
        compiler_params=pltpu.CompilerParams(needs_layout_passes=False),
        scratch_types=[
            pltpu.VMEM((ROWS_D,), jnp.int32),
            pltpu.VMEM((ROWS_D, D), F32),
            pltpu.SemaphoreType.DMA,
        ],
    )(src, xln)


def _head_krn(t_ref, sm_ref, eo_ref, g_ref, b_ref, hw_ref, hb_ref,
              out_ref):
    y0 = lax.dot_general(sm_ref[0, 0:1, :], eo_ref[...],
                         (((1,), (0,)), ((), ())),
                         preferred_element_type=F32)
    x = t_ref[0, 0:1, :] + y0
    xln = _ln(x, g_ref[...], b_ref[...])
    out_ref[0] = _bdot(xln, hw_ref[...]) + hb_ref[...]


def _head(t, sm, eo, params):
    out = pl.pallas_call(
        _head_krn,
        grid=(B,),
        in_specs=[
            pl.BlockSpec((1, 8, D), lambda b: (b, 0, 0)),
            pl.BlockSpec((1, 8, EC), lambda b: (b, 0, 0)),
            pl.BlockSpec((EC, D), lambda b: (0, 0)),
            pl.BlockSpec((D,), lambda b: (0,)),
            pl.BlockSpec((D,), lambda b: (0,)),
            pl.BlockSpec((D, 1000), lambda b: (0, 0)),
            pl.BlockSpec((1000,), lambda b: (0,)),
        ],
        out_specs=pl.BlockSpec((1, 1, 1000), lambda b: (b, 0, 0)),
        out_shape=jax.ShapeDtypeStruct((B, 1, 1000), F32),
    )(t, sm.reshape(B, NP, EC), eo,
      params['ln_g'], params['ln_b'], params['head_w'], params['head_b'])
    return out.reshape(B, 1000)


def kernel(x, params):
    GP, PATCH = 14, 16
    patches = x.reshape(B, 3, GP, PATCH, GP, PATCH)
    patches = patches.transpose(0, 2, 4, 1, 3, 5).reshape(
        B, GP * GP, 3 * PATCH * PATCH)
    pos = params['pos'][0]
    pos_pad = jnp.concatenate([pos, jnp.zeros((NP - NV, D), F32)], axis=0)
    cls_vec = params['cls'].reshape(1, D)
    t = _embed(patches, params['patch_w'], params['patch_b'], cls_vec,
               pos_pad)

    sm, eo = None, None
    for p in params['blocks']:
        t = _attn_block(t, p, sm, eo)
        sm, eo = None, None
        if 'gate_w' in p:
            xln, src, sm = _route(t, p)
            buf = _sc_dispatch(src, xln)
            eo = _experts(buf, p)
        else:
            t = _mlp_block(t, p)
    return _head(t, sm, eo, params)
